# baseline (speedup 1.0000x reference)
import functools

import jax
import jax.numpy as jnp
from jax import lax
from jax.experimental import pallas as pl
from jax.experimental.pallas import tpu as pltpu
from jax.experimental.pallas import tpu_sc as plsc

_NC, _NS = 2, 16
_NW = _NC * _NS
_LANES = 16
_K_OUT = 100
_IDX_CHUNK = 128


def _sc_dedup_body(T, topk_hbm, emb_hbm, out_hbm, idx2d, pids_v, table, sem):
    w = lax.axis_index("s") * _NC + lax.axis_index("c")
    pltpu.sync_copy(topk_hbm.at[w], idx2d)
    n_chunks = T // _IDX_CHUNK
    descs = [
        pltpu.async_copy(
            emb_hbm.at[idx2d.at[j]],
            pids_v.at[pl.ds(j * _IDX_CHUNK, _IDX_CHUNK)],
            sem,
        )
        for j in range(n_chunks)
    ]
    for d in descs:
        d.wait()
    base = lax.iota(jnp.int32, _LANES)
    for c in range(T // _LANES):
        p = pids_v[pl.ds(c * _LANES, _LANES)]
        plsc.store_scatter(table, [p], c * _LANES + base)
    for c in range(T // _LANES):
        jv = c * _LANES + base
        p = pids_v[pl.ds(c * _LANES, _LANES)]
        winner = plsc.load_gather(table, [p])
        pids_v[pl.ds(c * _LANES, _LANES)] = jnp.where(winner == jv, p, -1)
    pltpu.sync_copy(pids_v, out_hbm.at[w])


def _sc_gather_scores_body(T, pids_hbm, scores_hbm, out_hbm, pids_v, row_v,
                           out_v, sem):
    w = lax.axis_index("s") * _NC + lax.axis_index("c")
    d1 = pltpu.async_copy(pids_hbm.at[w], pids_v, sem)
    d2 = pltpu.async_copy(scores_hbm.at[w], row_v, sem)
    d1.wait()
    d2.wait()
    neg_inf = jnp.float32(-jnp.inf)
    for c in range(T // _LANES):
        p = pids_v[pl.ds(c * _LANES, _LANES)]
        s = plsc.load_gather(row_v, [jnp.maximum(p, 0)])
        out_v[pl.ds(c * _LANES, _LANES)] = jnp.where(p < 0, neg_inf, s)
    pltpu.sync_copy(out_v, out_hbm.at[w])


def _tc_scores_body(B, Qn, D, q_ref, v_ref, o_ref, s_buf):
    q = q_ref[...]
    half = D // 2
    acc = None
    for g in range(2):
        for t in range(half):
            s_buf[t] = lax.dot_general(
                q, v_ref[:, g * half + t, :], (((1,), (1,)), ((), ())))
        m = jnp.max(s_buf[...], axis=0)
        acc = m if acc is None else jnp.maximum(acc, m)
    pb = acc.shape[1]
    o_ref[...] = jnp.mean(acc.reshape(B, Qn, pb), axis=1)


def _tc_topk_body(T, s_ref, p_ref, op_ref, os_ref):
    s = s_ref[...]
    pids = p_ref[...]
    neg_inf = jnp.float32(-jnp.inf)
    big = jnp.int32(2 ** 30)
    for i in range(_K_OUT):
        m = jnp.max(s, axis=1, keepdims=True)
        eq = s == m
        pid_sel = jnp.min(jnp.where(eq, pids, big), axis=1, keepdims=True)
        os_ref[:, pl.ds(i, 1)] = m
        op_ref[:, pl.ds(i, 1)] = pid_sel
        s = jnp.where(eq & (pids == pid_sel), neg_inf, s)


def kernel(q_vectors, topk_indices, vectors, emb2pid, k):
    B, Qn, H = q_vectors.shape
    T = topk_indices.shape[1]
    NP, D, _ = vectors.shape
    mesh = plsc.VectorSubcoreMesh(core_axis_name="c", subcore_axis_name="s",
                                  num_cores=_NC, num_subcores=_NS)

    sc_dedup = pl.kernel(
        functools.partial(_sc_dedup_body, T),
        out_type=jax.ShapeDtypeStruct((B, T), jnp.int32),
        mesh=mesh,
        compiler_params=pltpu.CompilerParams(needs_layout_passes=False),
        scratch_types=[
            pltpu.VMEM((T // _IDX_CHUNK, _IDX_CHUNK), jnp.int32),
            pltpu.VMEM((T,), jnp.int32),
            pltpu.VMEM((NP,), jnp.int32),
            pltpu.SemaphoreType.DMA,
        ],
    )
    cand_pids = sc_dedup(topk_indices.reshape(B, T // _IDX_CHUNK, _IDX_CHUNK),
                         emb2pid)

    pid_block = 512
    np_pad = 10240
    scores_all = pl.pallas_call(
        functools.partial(_tc_scores_body, B, Qn, D),
        grid=(np_pad // pid_block,),
        in_specs=[
            pl.BlockSpec((B * Qn, H), lambda i: (0, 0)),
            pl.BlockSpec((pid_block, D, H), lambda i: (i, 0, 0)),
        ],
        out_specs=pl.BlockSpec((B, pid_block), lambda i: (0, i)),
        out_shape=jax.ShapeDtypeStruct((B, np_pad), jnp.float32),
        scratch_shapes=[pltpu.VMEM((D // 2, B * Qn, pid_block), jnp.float32)],
    )(q_vectors.reshape(B * Qn, H), vectors)

    sc_gather = pl.kernel(
        functools.partial(_sc_gather_scores_body, T),
        out_type=jax.ShapeDtypeStruct((B, T), jnp.float32),
        mesh=mesh,
        compiler_params=pltpu.CompilerParams(needs_layout_passes=False),
        scratch_types=[
            pltpu.VMEM((T,), jnp.int32),
            pltpu.VMEM((np_pad,), jnp.float32),
            pltpu.VMEM((T,), jnp.float32),
            pltpu.SemaphoreType.DMA,
        ],
    )
    cand_scores = sc_gather(cand_pids, scores_all)

    maxsim_pids, maxsim_scores = pl.pallas_call(
        functools.partial(_tc_topk_body, T),
        out_shape=(
            jax.ShapeDtypeStruct((B, _K_OUT), jnp.int32),
            jax.ShapeDtypeStruct((B, _K_OUT), jnp.float32),
        ),
    )(cand_scores, cand_pids)
    return maxsim_pids, maxsim_scores


# speedup vs baseline: 1.0866x; 1.0866x over previous
import functools

import jax
import jax.numpy as jnp
from jax import lax
from jax.experimental import pallas as pl
from jax.experimental.pallas import tpu as pltpu
from jax.experimental.pallas import tpu_sc as plsc

_NC, _NS = 2, 16
_NW = _NC * _NS
_LANES = 16
_K_OUT = 100
_IDX_CHUNK = 128


def _sc_dedup_body(T, topk_hbm, emb_hbm, out_hbm, idx2d, pids_v, table, sem):
    w = lax.axis_index("s") * _NC + lax.axis_index("c")
    pltpu.sync_copy(topk_hbm.at[w], idx2d)
    n_chunks = T // _IDX_CHUNK
    descs = [
        pltpu.async_copy(
            emb_hbm.at[idx2d.at[j]],
            pids_v.at[pl.ds(j * _IDX_CHUNK, _IDX_CHUNK)],
            sem,
        )
        for j in range(n_chunks)
    ]
    for d in descs:
        d.wait()
    base = lax.iota(jnp.int32, _LANES)
    for c in range(T // _LANES):
        p = pids_v[pl.ds(c * _LANES, _LANES)]
        plsc.store_scatter(table, [p], c * _LANES + base)
    for c in range(T // _LANES):
        jv = c * _LANES + base
        p = pids_v[pl.ds(c * _LANES, _LANES)]
        winner = plsc.load_gather(table, [p])
        pids_v[pl.ds(c * _LANES, _LANES)] = jnp.where(winner == jv, p, -1)
    pltpu.sync_copy(pids_v, out_hbm.at[w])


def _sc_gather_scores_body(T, pids_hbm, scores_hbm, out_hbm, pids_v, row_v,
                           out_v, sem):
    w = lax.axis_index("s") * _NC + lax.axis_index("c")
    d1 = pltpu.async_copy(pids_hbm.at[w], pids_v, sem)
    d2 = pltpu.async_copy(scores_hbm.at[w], row_v, sem)
    d1.wait()
    d2.wait()
    neg_inf = jnp.float32(-jnp.inf)
    for c in range(T // _LANES):
        p = pids_v[pl.ds(c * _LANES, _LANES)]
        s = plsc.load_gather(row_v, [jnp.maximum(p, 0)])
        out_v[pl.ds(c * _LANES, _LANES)] = jnp.where(p < 0, neg_inf, s)
    pltpu.sync_copy(out_v, out_hbm.at[w])


def _tc_scores_body(B, Qn, D, q_ref, v_ref, o_ref, s_buf):
    q = q_ref[...]
    for d in range(D):
        s_buf[d] = lax.dot_general(q, v_ref[:, d, :], (((1,), (1,)), ((), ())))
    acc = jnp.max(s_buf[...], axis=0)
    pb = acc.shape[1]
    o_ref[...] = jnp.mean(acc.reshape(B, Qn, pb), axis=1)


def _tc_topk_body(T, s_ref, p_ref, op_ref, os_ref):
    s = s_ref[...]
    pids = p_ref[...]
    neg_inf = jnp.float32(-jnp.inf)
    big = jnp.int32(2 ** 30)
    for i in range(_K_OUT):
        m = jnp.max(s, axis=1, keepdims=True)
        eq = s == m
        pid_sel = jnp.min(jnp.where(eq, pids, big), axis=1, keepdims=True)
        os_ref[:, pl.ds(i, 1)] = m
        op_ref[:, pl.ds(i, 1)] = pid_sel
        s = jnp.where(eq & (pids == pid_sel), neg_inf, s)


def kernel(q_vectors, topk_indices, vectors, emb2pid, k):
    B, Qn, H = q_vectors.shape
    T = topk_indices.shape[1]
    NP, D, _ = vectors.shape
    mesh = plsc.VectorSubcoreMesh(core_axis_name="c", subcore_axis_name="s",
                                  num_cores=_NC, num_subcores=_NS)

    sc_dedup = pl.kernel(
        functools.partial(_sc_dedup_body, T),
        out_type=jax.ShapeDtypeStruct((B, T), jnp.int32),
        mesh=mesh,
        compiler_params=pltpu.CompilerParams(needs_layout_passes=False),
        scratch_types=[
            pltpu.VMEM((T // _IDX_CHUNK, _IDX_CHUNK), jnp.int32),
            pltpu.VMEM((T,), jnp.int32),
            pltpu.VMEM((NP,), jnp.int32),
            pltpu.SemaphoreType.DMA,
        ],
    )
    cand_pids = sc_dedup(topk_indices.reshape(B, T // _IDX_CHUNK, _IDX_CHUNK),
                         emb2pid)

    pid_block = 256
    np_pad = 10240
    scores_all = pl.pallas_call(
        functools.partial(_tc_scores_body, B, Qn, D),
        grid=(np_pad // pid_block,),
        in_specs=[
            pl.BlockSpec((B * Qn, H), lambda i: (0, 0)),
            pl.BlockSpec((pid_block, D, H), lambda i: (i, 0, 0)),
        ],
        out_specs=pl.BlockSpec((B, pid_block), lambda i: (0, i)),
        out_shape=jax.ShapeDtypeStruct((B, np_pad), jnp.float32),
        scratch_shapes=[pltpu.VMEM((D, B * Qn, pid_block), jnp.float32)],
    )(q_vectors.reshape(B * Qn, H), vectors)

    sc_gather = pl.kernel(
        functools.partial(_sc_gather_scores_body, T),
        out_type=jax.ShapeDtypeStruct((B, T), jnp.float32),
        mesh=mesh,
        compiler_params=pltpu.CompilerParams(needs_layout_passes=False),
        scratch_types=[
            pltpu.VMEM((T,), jnp.int32),
            pltpu.VMEM((np_pad,), jnp.float32),
            pltpu.VMEM((T,), jnp.float32),
            pltpu.SemaphoreType.DMA,
        ],
    )
    cand_scores = sc_gather(cand_pids, scores_all)

    maxsim_pids, maxsim_scores = pl.pallas_call(
        functools.partial(_tc_topk_body, T),
        out_shape=(
            jax.ShapeDtypeStruct((B, _K_OUT), jnp.int32),
            jax.ShapeDtypeStruct((B, _K_OUT), jnp.float32),
        ),
    )(cand_scores, cand_pids)
    return maxsim_pids, maxsim_scores


# speedup vs baseline: 1.0875x; 1.0008x over previous
import functools

import jax
import jax.numpy as jnp
from jax import lax
from jax.experimental import pallas as pl
from jax.experimental.pallas import tpu as pltpu
from jax.experimental.pallas import tpu_sc as plsc

_NC, _NS = 2, 16
_LANES = 16
_K_OUT = 100
_IDX_CHUNK = 128


def _sc_dedup_body(T, topk_hbm, emb_hbm, out_hbm, idx2d, pids_v, table, sem):
    w = lax.axis_index("s") * _NC + lax.axis_index("c")
    pltpu.sync_copy(topk_hbm.at[w], idx2d)
    n_chunks = T // _IDX_CHUNK
    descs = [
        pltpu.async_copy(
            emb_hbm.at[idx2d.at[j]],
            pids_v.at[pl.ds(j * _IDX_CHUNK, _IDX_CHUNK)],
            sem,
        )
        for j in range(n_chunks)
    ]
    for d in descs:
        d.wait()
    base = lax.iota(jnp.int32, _LANES)
    for c in range(T // _LANES):
        p = pids_v[pl.ds(c * _LANES, _LANES)]
        plsc.store_scatter(table, [p], c * _LANES + base)
    for c in range(T // _LANES):
        jv = c * _LANES + base
        p = pids_v[pl.ds(c * _LANES, _LANES)]
        winner = plsc.load_gather(table, [p])
        pids_v[pl.ds(c * _LANES, _LANES)] = jnp.where(winner == jv, p, -1)
    pltpu.sync_copy(pids_v, out_hbm.at[w])


def _sc_gather_scores_body(T, pids_hbm, scores_hbm, out_hbm, pids_v, row_v,
                           out_v, sem):
    w = lax.axis_index("s") * _NC + lax.axis_index("c")
    d1 = pltpu.async_copy(pids_hbm.at[w], pids_v, sem)
    d2 = pltpu.async_copy(scores_hbm.at[w], row_v, sem)
    d1.wait()
    d2.wait()
    neg_inf = jnp.float32(-jnp.inf)
    for c in range(T // _LANES):
        p = pids_v[pl.ds(c * _LANES, _LANES)]
        s = plsc.load_gather(row_v, [jnp.maximum(p, 0)])
        out_v[pl.ds(c * _LANES, _LANES)] = jnp.where(p < 0, neg_inf, s)
    pltpu.sync_copy(out_v, out_hbm.at[w])


def _tc_scores_body(B, Qn, D, q_ref, v_ref, o_ref, s_buf):
    q = q_ref[...]
    for d in range(D):
        s_buf[d] = lax.dot_general(q, v_ref[:, d, :], (((1,), (1,)), ((), ())))
    acc = jnp.max(s_buf[...], axis=0)
    pb = acc.shape[1]
    o_ref[...] = jnp.mean(acc.reshape(B, Qn, pb), axis=1)


def _tc_topk_body(T, s_ref, p_ref, op_ref, os_ref):
    s = s_ref[...]
    pids = p_ref[...]
    neg_inf = jnp.float32(-jnp.inf)
    big = jnp.int32(2 ** 30)
    for i in range(_K_OUT):
        m = jnp.max(s, axis=1, keepdims=True)
        eq = s == m
        pid_sel = jnp.min(jnp.where(eq, pids, big), axis=1, keepdims=True)
        os_ref[:, pl.ds(i, 1)] = m
        op_ref[:, pl.ds(i, 1)] = pid_sel
        s = jnp.where(eq & (pids == pid_sel), neg_inf, s)


def kernel(q_vectors, topk_indices, vectors, emb2pid, k):
    B, Qn, H = q_vectors.shape
    T = topk_indices.shape[1]
    NP, D, _ = vectors.shape
    mesh = plsc.VectorSubcoreMesh(core_axis_name="c", subcore_axis_name="s",
                                  num_cores=_NC, num_subcores=_NS)

    sc_dedup = pl.kernel(
        functools.partial(_sc_dedup_body, T),
        out_type=jax.ShapeDtypeStruct((B, T), jnp.int32),
        mesh=mesh,
        compiler_params=pltpu.CompilerParams(needs_layout_passes=False),
        scratch_types=[
            pltpu.VMEM((T // _IDX_CHUNK, _IDX_CHUNK), jnp.int32),
            pltpu.VMEM((T,), jnp.int32),
            pltpu.VMEM((NP,), jnp.int32),
            pltpu.SemaphoreType.DMA,
        ],
    )
    cand_pids = sc_dedup(topk_indices.reshape(B, T // _IDX_CHUNK, _IDX_CHUNK),
                         emb2pid)

    pid_block = 256
    np_pad = 10240
    scores_all = pl.pallas_call(
        functools.partial(_tc_scores_body, B, Qn, D),
        grid=(np_pad // pid_block,),
        in_specs=[
            pl.BlockSpec((B * Qn, H), lambda i: (0, 0)),
            pl.BlockSpec((pid_block, D, H), lambda i: (i, 0, 0)),
        ],
        out_specs=pl.BlockSpec((B, pid_block), lambda i: (0, i)),
        out_shape=jax.ShapeDtypeStruct((B, np_pad), jnp.float32),
        scratch_shapes=[pltpu.VMEM((D, B * Qn, pid_block), jnp.float32)],
    )(q_vectors.reshape(B * Qn, H), vectors)

    sc_gather = pl.kernel(
        functools.partial(_sc_gather_scores_body, T),
        out_type=jax.ShapeDtypeStruct((B, T), jnp.float32),
        mesh=mesh,
        compiler_params=pltpu.CompilerParams(needs_layout_passes=False),
        scratch_types=[
            pltpu.VMEM((T,), jnp.int32),
            pltpu.VMEM((np_pad,), jnp.float32),
            pltpu.VMEM((T,), jnp.float32),
            pltpu.SemaphoreType.DMA,
        ],
    )
    cand_scores = sc_gather(cand_pids, scores_all)

    maxsim_pids, maxsim_scores = pl.pallas_call(
        functools.partial(_tc_topk_body, T),
        out_shape=(
            jax.ShapeDtypeStruct((B, _K_OUT), jnp.int32),
            jax.ShapeDtypeStruct((B, _K_OUT), jnp.float32),
        ),
    )(cand_scores, cand_pids)
    return maxsim_pids, maxsim_scores
